# Initial kernel scaffold; baseline (speedup 1.0000x reference)
import jax
import jax.numpy as jnp
from jax.experimental import pallas as pl


def kernel(compound_diseases, compound_phenotypes, compound_subcellular_locations, protein_diseases, protein_phenotypes, protein_subcellular_locations, disease_table, phenotype_table, sub_table, W1, b1, W2, b2, W3, b3):
    raise NotImplementedError("write your pallas kernel here")


# speedup vs baseline: 27.3449x; 27.3449x over previous
import functools

import jax
import jax.numpy as jnp
import numpy as np
from jax import lax
from jax.experimental import pallas as pl
from jax.experimental.pallas import tpu as pltpu
from jax.experimental.pallas import tpu_sc as plsc

B = 16384
L = 200
LS = 20
ND, NP, NS = 13752, 17393, 30
DD, DP, DS = 32, 16, 16
IN = (DD + DP + DS) * 2
H1, H2 = 128, 64

NW = 32
RW = B // NW
TR = 8
NT = RW // TR
W = 80
NCH = TR * L // W
NCHS = TR * LS // W

_f32 = jnp.float32
_i32 = jnp.int32


def _sc_pool(cd2, pd2, cp2, pp2, cs2, ps2, dis_t, phe_t, sub_t,
             seg_b, seg_bs, zer32, zer16,
             o_cd, o_pd, o_cp, o_pp, o_cs, o_ps,
             idx_v, idx_vs, seg_v, seg_vs, rows32, rows16, rows16s,
             sh_d, sh_p, sh_s, sem):
  c = lax.axis_index("c")
  s = lax.axis_index("s")
  wid = c * 16 + s
  row0 = wid * RW
  sbase = s * RW

  def seg_init():
    pltpu.sync_copy(seg_b, seg_v)
    pltpu.sync_copy(seg_bs, seg_vs)

    @pl.loop(0, NCH)
    def _(j):
      for k in range(W // 16):
        sl = pl.ds(k * 16, 16)
        seg_v[j, sl] = seg_v[j, sl] + jnp.full((16,), sbase, _i32)

    @pl.loop(0, NCHS)
    def _(j):
      for k in range(W // 16):
        sl = pl.ds(k * 16, 16)
        seg_vs[j, sl] = seg_vs[j, sl] + jnp.full((16,), sbase, _i32)

  def do_group(table, acc, idx_buf, seg_buf, rows, nch):
    copies = []
    for j in range(nch):
      copies.append(pltpu.async_copy(
          table.at[idx_buf.at[pl.ds(j * W, W)]], rows.at[pl.ds(j * W, W)],
          sem))
    for cp in copies:
      cp.wait()
    for j in range(nch):
      pltpu.sync_copy(rows.at[pl.ds(j * W, W)], acc.at[seg_buf.at[j]],
                      add=True)

  def one_pass(ixd, ixp, ixs, o_d, o_p, o_s):
    pltpu.sync_copy(zer32, sh_d.at[pl.ds(sbase, RW)])
    pltpu.sync_copy(zer16, sh_p.at[pl.ds(sbase, RW)])
    pltpu.sync_copy(zer16, sh_s.at[pl.ds(sbase, RW)])
    seg_init()

    @pl.loop(0, NT)
    def _(t):
      base = row0 + t * TR
      ib = base * L
      ibs = base * LS
      pltpu.sync_copy(ixd.at[pl.ds(ib, TR * L)], idx_v)
      do_group(dis_t, sh_d, idx_v, seg_v, rows32, NCH)
      pltpu.sync_copy(ixp.at[pl.ds(ib, TR * L)], idx_v)
      do_group(phe_t, sh_p, idx_v, seg_v, rows16, NCH)
      pltpu.sync_copy(ixs.at[pl.ds(ibs, TR * LS)], idx_vs)
      do_group(sub_t, sh_s, idx_vs, seg_vs, rows16s, NCHS)

      @pl.loop(0, NCH)
      def _(j):
        for k in range(W // 16):
          sl = pl.ds(k * 16, 16)
          seg_v[j, sl] = seg_v[j, sl] + jnp.full((16,), TR, _i32)

      @pl.loop(0, NCHS)
      def _(j):
        for k in range(W // 16):
          sl = pl.ds(k * 16, 16)
          seg_vs[j, sl] = seg_vs[j, sl] + jnp.full((16,), TR, _i32)

    pltpu.sync_copy(sh_d.at[pl.ds(sbase, RW)], o_d.at[pl.ds(row0, RW)])
    pltpu.sync_copy(sh_p.at[pl.ds(sbase, RW)], o_p.at[pl.ds(row0, RW)])
    pltpu.sync_copy(sh_s.at[pl.ds(sbase, RW)], o_s.at[pl.ds(row0, RW)])

  one_pass(cd2, cp2, cs2, o_cd, o_cp, o_cs)
  one_pass(pd2, pp2, ps2, o_pd, o_pp, o_ps)


def _mlp_body(cd_r, cp_r, cs_r, pd_r, pp_r, ps_r,
              w1_r, b1_r, w2_r, b2_r, w3_r, b3_r, o_r):
  inv_l = np.float32(1.0 / L)
  inv_ls = np.float32(1.0 / LS)
  x = jnp.concatenate(
      [cd_r[...] * inv_l, cp_r[...] * inv_l, cs_r[...] * inv_ls,
       pd_r[...] * inv_l, pp_r[...] * inv_l, ps_r[...] * inv_ls], axis=1)
  hp = jax.lax.Precision.HIGHEST
  h = jnp.dot(x, w1_r[...], preferred_element_type=_f32, precision=hp)
  h = h + b1_r[...]
  h = jnp.where(h >= 0, h, h * np.float32(0.01))
  h = jnp.dot(h, w2_r[...], preferred_element_type=_f32, precision=hp)
  h = h + b2_r[...]
  h = jnp.where(h >= 0, h, h * np.float32(0.01))
  o = jnp.dot(h, w3_r[...], preferred_element_type=_f32, precision=hp)
  o_r[...] = o + b3_r[...]


def kernel(compound_diseases, compound_phenotypes, compound_subcellular_locations,
           protein_diseases, protein_phenotypes, protein_subcellular_locations,
           disease_table, phenotype_table, sub_table, W1, b1, W2, b2, W3, b3):
  cd2 = compound_diseases.astype(_i32).reshape(B * L)
  pd2 = protein_diseases.astype(_i32).reshape(B * L)
  cp2 = compound_phenotypes.astype(_i32).reshape(B * L)
  pp2 = protein_phenotypes.astype(_i32).reshape(B * L)
  cs2 = compound_subcellular_locations.astype(_i32).reshape(B * LS)
  ps2 = protein_subcellular_locations.astype(_i32).reshape(B * LS)

  seg_b = jnp.asarray(np.arange(TR * L, dtype=np.int32).reshape(NCH, W) // L)
  seg_bs = jnp.asarray(np.arange(TR * LS, dtype=np.int32).reshape(NCHS, W) // LS)
  zer32 = jnp.zeros((RW, DD), _f32)
  zer16 = jnp.zeros((RW, DP), _f32)

  mesh = plsc.VectorSubcoreMesh(core_axis_name="c", subcore_axis_name="s")
  acc32 = jax.ShapeDtypeStruct((B, DD), _f32)
  acc16 = jax.ShapeDtypeStruct((B, DP), _f32)

  pool = pl.kernel(
      _sc_pool,
      out_type=(acc32, acc32, acc16, acc16, acc16, acc16),
      mesh=mesh,
      scratch_types=[
          pltpu.VMEM((TR * L,), _i32),
          pltpu.VMEM((TR * LS,), _i32),
          pltpu.VMEM((NCH, W), _i32),
          pltpu.VMEM((NCHS, W), _i32),
          pltpu.VMEM((TR * L, DD), _f32),
          pltpu.VMEM((TR * L, DP), _f32),
          pltpu.VMEM((TR * LS, DS), _f32),
          pltpu.VMEM_SHARED((16 * RW, DD), _f32),
          pltpu.VMEM_SHARED((16 * RW, DP), _f32),
          pltpu.VMEM_SHARED((16 * RW, DS), _f32),
          pltpu.SemaphoreType.DMA,
      ],
      compiler_params=pltpu.CompilerParams(use_tc_tiling_on_sc=False),
  )
  s_cd, s_pd, s_cp, s_pp, s_cs, s_ps = pool(
      cd2, pd2, cp2, pp2, cs2, ps2, disease_table, phenotype_table, sub_table,
      seg_b, seg_bs, zer32, zer16)

  bm = 512
  grid = (B // bm,)
  mlp = pl.pallas_call(
      _mlp_body,
      grid=grid,
      in_specs=[
          pl.BlockSpec((bm, DD), lambda i: (i, 0)),
          pl.BlockSpec((bm, DP), lambda i: (i, 0)),
          pl.BlockSpec((bm, DS), lambda i: (i, 0)),
          pl.BlockSpec((bm, DD), lambda i: (i, 0)),
          pl.BlockSpec((bm, DP), lambda i: (i, 0)),
          pl.BlockSpec((bm, DS), lambda i: (i, 0)),
          pl.BlockSpec((IN, H1), lambda i: (0, 0)),
          pl.BlockSpec((1, H1), lambda i: (0, 0)),
          pl.BlockSpec((H1, H2), lambda i: (0, 0)),
          pl.BlockSpec((1, H2), lambda i: (0, 0)),
          pl.BlockSpec((H2, 1), lambda i: (0, 0)),
          pl.BlockSpec((1, 1), lambda i: (0, 0)),
      ],
      out_specs=pl.BlockSpec((bm, 1), lambda i: (i, 0)),
      out_shape=jax.ShapeDtypeStruct((B, 1), _f32),
  )
  return mlp(s_cd, s_cp, s_cs, s_pd, s_pp, s_ps,
             W1, b1.reshape(1, H1), W2, b2.reshape(1, H2),
             W3, b3.reshape(1, 1))


# speedup vs baseline: 33.5593x; 1.2273x over previous
import functools

import jax
import jax.numpy as jnp
import numpy as np
from jax import lax
from jax.experimental import pallas as pl
from jax.experimental.pallas import tpu as pltpu
from jax.experimental.pallas import tpu_sc as plsc

B = 16384
L = 200
LS = 20
ND, NP, NS = 13752, 17393, 30
DD, DP, DS = 32, 16, 16
IN = (DD + DP + DS) * 2
H1, H2 = 128, 64

NW = 32
RW = B // NW
TR = 8
NT = RW // TR
W = 80
NCH = TR * L // W
NCHS = TR * LS // W

_f32 = jnp.float32
_i32 = jnp.int32


def _sc_pool(cd2, pd2, cp2, pp2, cs2, ps2, dis_t, phe_t, sub_t,
             seg_b, seg_bs, zer32, zer16,
             o_cd, o_pd, o_cp, o_pp, o_cs, o_ps,
             idx_v, idx_v2, idx_vs, seg_v, seg_vs, rows32, rows16, rows16s,
             sh_d, sh_p, sh_s, sem):
  c = lax.axis_index("c")
  s = lax.axis_index("s")
  wid = c * 16 + s
  row0 = wid * RW
  sbase = s * RW

  def seg_init():
    pltpu.sync_copy(seg_b, seg_v)
    pltpu.sync_copy(seg_bs, seg_vs)

    @pl.loop(0, NCH)
    def _(j):
      for k in range(W // 16):
        sl = pl.ds(k * 16, 16)
        seg_v[j, sl] = seg_v[j, sl] + jnp.full((16,), sbase, _i32)

    @pl.loop(0, NCHS)
    def _(j):
      for k in range(W // 16):
        sl = pl.ds(k * 16, 16)
        seg_vs[j, sl] = seg_vs[j, sl] + jnp.full((16,), sbase, _i32)

  def fire_gathers(table, idx_buf, rows, nch):
    copies = []
    for j in range(nch):
      copies.append(pltpu.async_copy(
          table.at[idx_buf.at[pl.ds(j * W, W)]], rows.at[pl.ds(j * W, W)],
          sem))
    return copies

  def fire_adds(acc, seg_buf, rows, nch):
    copies = []
    for j in range(nch):
      copies.append(pltpu.async_copy(
          rows.at[pl.ds(j * W, W)], acc.at[seg_buf.at[j]], sem, add=True))
    return copies

  def one_pass(ixd, ixp, ixs, o_d, o_p, o_s):
    pltpu.sync_copy(zer32, sh_d.at[pl.ds(sbase, RW)])
    pltpu.sync_copy(zer16, sh_p.at[pl.ds(sbase, RW)])
    pltpu.sync_copy(zer16, sh_s.at[pl.ds(sbase, RW)])
    seg_init()

    @pl.loop(0, NT)
    def _(t):
      base = row0 + t * TR
      ib = base * L
      ibs = base * LS
      ic = [pltpu.async_copy(ixd.at[pl.ds(ib, TR * L)], idx_v, sem),
            pltpu.async_copy(ixp.at[pl.ds(ib, TR * L)], idx_v2, sem),
            pltpu.async_copy(ixs.at[pl.ds(ibs, TR * LS)], idx_vs, sem)]
      for cp in ic:
        cp.wait()
      gc = (fire_gathers(dis_t, idx_v, rows32, NCH)
            + fire_gathers(phe_t, idx_v2, rows16, NCH)
            + fire_gathers(sub_t, idx_vs, rows16s, NCHS))
      for cp in gc:
        cp.wait()
      ac = (fire_adds(sh_d, seg_v, rows32, NCH)
            + fire_adds(sh_p, seg_v, rows16, NCH)
            + fire_adds(sh_s, seg_vs, rows16s, NCHS))
      for cp in ac:
        cp.wait()

      @pl.loop(0, NCH)
      def _(j):
        for k in range(W // 16):
          sl = pl.ds(k * 16, 16)
          seg_v[j, sl] = seg_v[j, sl] + jnp.full((16,), TR, _i32)

      @pl.loop(0, NCHS)
      def _(j):
        for k in range(W // 16):
          sl = pl.ds(k * 16, 16)
          seg_vs[j, sl] = seg_vs[j, sl] + jnp.full((16,), TR, _i32)

    pltpu.sync_copy(sh_d.at[pl.ds(sbase, RW)], o_d.at[pl.ds(row0, RW)])
    pltpu.sync_copy(sh_p.at[pl.ds(sbase, RW)], o_p.at[pl.ds(row0, RW)])
    pltpu.sync_copy(sh_s.at[pl.ds(sbase, RW)], o_s.at[pl.ds(row0, RW)])

  one_pass(cd2, cp2, cs2, o_cd, o_cp, o_cs)
  one_pass(pd2, pp2, ps2, o_pd, o_pp, o_ps)


def _mlp_body(cd_r, cp_r, cs_r, pd_r, pp_r, ps_r,
              w1_r, b1_r, w2_r, b2_r, w3_r, b3_r, o_r):
  inv_l = np.float32(1.0 / L)
  inv_ls = np.float32(1.0 / LS)
  x = jnp.concatenate(
      [cd_r[...] * inv_l, cp_r[...] * inv_l, cs_r[...] * inv_ls,
       pd_r[...] * inv_l, pp_r[...] * inv_l, ps_r[...] * inv_ls], axis=1)
  hp = jax.lax.Precision.HIGHEST
  h = jnp.dot(x, w1_r[...], preferred_element_type=_f32, precision=hp)
  h = h + b1_r[...]
  h = jnp.where(h >= 0, h, h * np.float32(0.01))
  h = jnp.dot(h, w2_r[...], preferred_element_type=_f32, precision=hp)
  h = h + b2_r[...]
  h = jnp.where(h >= 0, h, h * np.float32(0.01))
  o = jnp.dot(h, w3_r[...], preferred_element_type=_f32, precision=hp)
  o_r[...] = o + b3_r[...]


def kernel(compound_diseases, compound_phenotypes, compound_subcellular_locations,
           protein_diseases, protein_phenotypes, protein_subcellular_locations,
           disease_table, phenotype_table, sub_table, W1, b1, W2, b2, W3, b3):
  cd2 = compound_diseases.astype(_i32).reshape(B * L)
  pd2 = protein_diseases.astype(_i32).reshape(B * L)
  cp2 = compound_phenotypes.astype(_i32).reshape(B * L)
  pp2 = protein_phenotypes.astype(_i32).reshape(B * L)
  cs2 = compound_subcellular_locations.astype(_i32).reshape(B * LS)
  ps2 = protein_subcellular_locations.astype(_i32).reshape(B * LS)

  seg_b = jnp.asarray(np.arange(TR * L, dtype=np.int32).reshape(NCH, W) // L)
  seg_bs = jnp.asarray(np.arange(TR * LS, dtype=np.int32).reshape(NCHS, W) // LS)
  zer32 = jnp.zeros((RW, DD), _f32)
  zer16 = jnp.zeros((RW, DP), _f32)

  mesh = plsc.VectorSubcoreMesh(core_axis_name="c", subcore_axis_name="s")
  acc32 = jax.ShapeDtypeStruct((B, DD), _f32)
  acc16 = jax.ShapeDtypeStruct((B, DP), _f32)

  pool = pl.kernel(
      _sc_pool,
      out_type=(acc32, acc32, acc16, acc16, acc16, acc16),
      mesh=mesh,
      scratch_types=[
          pltpu.VMEM((TR * L,), _i32),
          pltpu.VMEM((TR * L,), _i32),
          pltpu.VMEM((TR * LS,), _i32),
          pltpu.VMEM((NCH, W), _i32),
          pltpu.VMEM((NCHS, W), _i32),
          pltpu.VMEM((TR * L, DD), _f32),
          pltpu.VMEM((TR * L, DP), _f32),
          pltpu.VMEM((TR * LS, DS), _f32),
          pltpu.VMEM_SHARED((16 * RW, DD), _f32),
          pltpu.VMEM_SHARED((16 * RW, DP), _f32),
          pltpu.VMEM_SHARED((16 * RW, DS), _f32),
          pltpu.SemaphoreType.DMA,
      ],
      compiler_params=pltpu.CompilerParams(use_tc_tiling_on_sc=False),
  )
  s_cd, s_pd, s_cp, s_pp, s_cs, s_ps = pool(
      cd2, pd2, cp2, pp2, cs2, ps2, disease_table, phenotype_table, sub_table,
      seg_b, seg_bs, zer32, zer16)

  bm = 512
  grid = (B // bm,)
  mlp = pl.pallas_call(
      _mlp_body,
      grid=grid,
      in_specs=[
          pl.BlockSpec((bm, DD), lambda i: (i, 0)),
          pl.BlockSpec((bm, DP), lambda i: (i, 0)),
          pl.BlockSpec((bm, DS), lambda i: (i, 0)),
          pl.BlockSpec((bm, DD), lambda i: (i, 0)),
          pl.BlockSpec((bm, DP), lambda i: (i, 0)),
          pl.BlockSpec((bm, DS), lambda i: (i, 0)),
          pl.BlockSpec((IN, H1), lambda i: (0, 0)),
          pl.BlockSpec((1, H1), lambda i: (0, 0)),
          pl.BlockSpec((H1, H2), lambda i: (0, 0)),
          pl.BlockSpec((1, H2), lambda i: (0, 0)),
          pl.BlockSpec((H2, 1), lambda i: (0, 0)),
          pl.BlockSpec((1, 1), lambda i: (0, 0)),
      ],
      out_specs=pl.BlockSpec((bm, 1), lambda i: (i, 0)),
      out_shape=jax.ShapeDtypeStruct((B, 1), _f32),
  )
  return mlp(s_cd, s_cp, s_cs, s_pd, s_pp, s_ps,
             W1, b1.reshape(1, H1), W2, b2.reshape(1, H2),
             W3, b3.reshape(1, 1))
